# baseline (speedup 1.0000x reference)
import functools

import jax
import jax.numpy as jnp
from jax import lax
from jax.experimental import pallas as pl
from jax.experimental.pallas import tpu as pltpu
from jax.experimental.pallas import tpu_sc as plsc

R, C = 16384, 200
D = 64
N = R * C

NC, NS = 2, 16
NW = NC * NS
PER_W = N // NW
CHUNK = 128
SUP = 256
GPC = SUP // CHUNK
NSUP = PER_W // SUP

_mesh = plsc.VectorSubcoreMesh(core_axis_name="c", subcore_axis_name="s")


@functools.partial(
    pl.kernel,
    mesh=_mesh,
    out_type=jax.ShapeDtypeStruct((N, D), jnp.float32),
    scratch_types=[
        pltpu.VMEM((SUP,), jnp.int32),
        pltpu.VMEM((SUP,), jnp.int32),
        pltpu.VMEM((SUP, D), jnp.float32),
        pltpu.VMEM((SUP, D), jnp.float32),
        pltpu.VMEM_SHARED((8, D), jnp.float32),
        pltpu.SemaphoreType.DMA,
        pltpu.SemaphoreType.DMA,
        pltpu.SemaphoreType.DMA,
        pltpu.SemaphoreType.DMA,
        pltpu.SemaphoreType.DMA,
    ],
    compiler_params=pltpu.CompilerParams(use_tc_tiling_on_sc=False),
)
def _emb_lookup(idx_hbm, tab_hbm, out_hbm, idx0, idx1, rows0, rows1, tab_v,
                s_idx0, s_idx1, s_gat, s_out0, s_out1):
    wid = lax.axis_index("s") * NC + lax.axis_index("c")
    w_base = wid * PER_W
    @pl.when(lax.axis_index("s") == 0)
    def _():
        pltpu.sync_copy(tab_hbm, tab_v)

    plsc.subcore_barrier()
    idx_b = (idx0, idx1)
    rows_b = (rows0, rows1)
    s_idx = (s_idx0, s_idx1)
    s_out = (s_out0, s_out1)

    def idx_start(g, b):
        pltpu.async_copy(idx_hbm.at[pl.ds(w_base + g * SUP, SUP)],
                         idx_b[b], s_idx[b])

    def idx_wait(b):
        pltpu.make_async_copy(idx_hbm.at[pl.ds(0, SUP)],
                              idx_b[b], s_idx[b]).wait()

    def out_start(g, b):
        pltpu.async_copy(rows_b[b],
                         out_hbm.at[pl.ds(w_base + g * SUP, SUP)], s_out[b])

    def out_wait(b):
        pltpu.make_async_copy(rows_b[b],
                              out_hbm.at[pl.ds(0, SUP)], s_out[b]).wait()

    idx_start(0, 0)
    idx_start(1, 1)

    def pair(p, carry):
        for b in range(2):
            g = 2 * p + b

            @pl.when(p > 0)
            def _():
                out_wait(b)

            idx_wait(b)
            for j in range(GPC):
                pltpu.async_copy(
                    tab_v.at[idx_b[b].at[pl.ds(j * CHUNK, CHUNK)]],
                    rows_b[b].at[pl.ds(j * CHUNK, CHUNK)], s_gat)
            for j in range(GPC):
                pltpu.make_async_copy(
                    tab_v.at[idx_b[b].at[pl.ds(0, CHUNK)]],
                    rows_b[b].at[pl.ds(0, CHUNK)], s_gat).wait()

            @pl.when(g + 2 < NSUP)
            def _():
                idx_start(g + 2, b)

            out_start(g, b)
        return carry

    lax.fori_loop(0, NSUP // 2, pair, 0)
    out_wait(0)
    out_wait(1)


def kernel(onehot, emb_weight):
    flat = onehot.reshape(N)
    out = _emb_lookup(flat, emb_weight)
    return out.reshape(R, C, D)


# speedup vs baseline: 2.1420x; 2.1420x over previous
import functools

import jax
import jax.numpy as jnp
from jax import lax
from jax.experimental import pallas as pl
from jax.experimental.pallas import tpu as pltpu
from jax.experimental.pallas import tpu_sc as plsc

R, C = 16384, 200
D = 64
N = R * C

NC, NS = 2, 16
NW = NC * NS
PER_W = N // NW
CHUNK = 128
SUP = 256
GPC = SUP // CHUNK
NSUP = PER_W // SUP

_mesh = plsc.VectorSubcoreMesh(core_axis_name="c", subcore_axis_name="s")


@functools.partial(
    pl.kernel,
    mesh=_mesh,
    out_type=jax.ShapeDtypeStruct((N, 128), jnp.float32),
    scratch_types=[
        pltpu.VMEM((SUP,), jnp.int32),
        pltpu.VMEM((SUP,), jnp.int32),
        pltpu.VMEM((SUP, D), jnp.float32),
        pltpu.VMEM((SUP, D), jnp.float32),
        pltpu.VMEM_SHARED((8, D), jnp.float32),
        pltpu.SemaphoreType.DMA,
        pltpu.SemaphoreType.DMA,
        pltpu.SemaphoreType.DMA,
        pltpu.SemaphoreType.DMA,
        pltpu.SemaphoreType.DMA,
    ],
    compiler_params=pltpu.CompilerParams(use_tc_tiling_on_sc=False),
)
def _emb_lookup(idx_hbm, tab_hbm, out_hbm, idx0, idx1, rows0, rows1, tab_v,
                s_idx0, s_idx1, s_gat, s_out0, s_out1):
    wid = lax.axis_index("s") * NC + lax.axis_index("c")
    w_base = wid * PER_W
    @pl.when(lax.axis_index("s") == 0)
    def _():
        pltpu.sync_copy(tab_hbm, tab_v)

    plsc.subcore_barrier()
    idx_b = (idx0, idx1)
    rows_b = (rows0, rows1)
    s_idx = (s_idx0, s_idx1)
    s_out = (s_out0, s_out1)

    def idx_start(g, b):
        pltpu.async_copy(idx_hbm.at[pl.ds(w_base + g * SUP, SUP)],
                         idx_b[b], s_idx[b])

    def idx_wait(b):
        pltpu.make_async_copy(idx_hbm.at[pl.ds(0, SUP)],
                              idx_b[b], s_idx[b]).wait()

    def out_start(g, b):
        pltpu.async_copy(rows_b[b],
                         out_hbm.at[pl.ds(w_base + g * SUP, SUP), pl.ds(0, D)],
                         s_out[b])

    def out_wait(b):
        pltpu.make_async_copy(rows_b[b],
                              out_hbm.at[pl.ds(0, SUP), pl.ds(0, D)],
                              s_out[b]).wait()

    idx_start(0, 0)
    idx_start(1, 1)

    def pair(p, carry):
        for b in range(2):
            g = 2 * p + b

            @pl.when(p > 0)
            def _():
                out_wait(b)

            idx_wait(b)
            for j in range(GPC):
                pltpu.async_copy(
                    tab_v.at[idx_b[b].at[pl.ds(j * CHUNK, CHUNK)]],
                    rows_b[b].at[pl.ds(j * CHUNK, CHUNK)], s_gat)
            for j in range(GPC):
                pltpu.make_async_copy(
                    tab_v.at[idx_b[b].at[pl.ds(0, CHUNK)]],
                    rows_b[b].at[pl.ds(0, CHUNK)], s_gat).wait()

            @pl.when(g + 2 < NSUP)
            def _():
                idx_start(g + 2, b)

            out_start(g, b)
        return carry

    lax.fori_loop(0, NSUP // 2, pair, 0)
    out_wait(0)
    out_wait(1)


def kernel(onehot, emb_weight):
    flat = onehot.reshape(N)
    out = _emb_lookup(flat, emb_weight)
    return out[:, :D].reshape(R, C, D)


# speedup vs baseline: 2.1705x; 1.0133x over previous
import functools

import jax
import jax.numpy as jnp
from jax import lax
from jax.experimental import pallas as pl
from jax.experimental.pallas import tpu as pltpu
from jax.experimental.pallas import tpu_sc as plsc

R, C = 16384, 200
D = 64
N = R * C

NC, NS = 2, 16
NW = NC * NS
PER_W = N // NW
CHUNK = 128
SUP = 512
GPC = SUP // CHUNK
NSUP = PER_W // SUP

_mesh = plsc.VectorSubcoreMesh(core_axis_name="c", subcore_axis_name="s")


@functools.partial(
    pl.kernel,
    mesh=_mesh,
    out_type=jax.ShapeDtypeStruct((N, 128), jnp.float32),
    scratch_types=[
        pltpu.VMEM((SUP,), jnp.int32),
        pltpu.VMEM((SUP,), jnp.int32),
        pltpu.VMEM((SUP, D), jnp.float32),
        pltpu.VMEM((SUP, D), jnp.float32),
        pltpu.VMEM_SHARED((8, D), jnp.float32),
        pltpu.SemaphoreType.DMA,
        pltpu.SemaphoreType.DMA,
        pltpu.SemaphoreType.DMA,
        pltpu.SemaphoreType.DMA,
        pltpu.SemaphoreType.DMA,
    ],
    compiler_params=pltpu.CompilerParams(use_tc_tiling_on_sc=False),
)
def _emb_lookup(idx_hbm, tab_hbm, out_hbm, idx0, idx1, rows0, rows1, tab_v,
                s_idx0, s_idx1, s_gat, s_out0, s_out1):
    wid = lax.axis_index("s") * NC + lax.axis_index("c")
    w_base = wid * PER_W
    @pl.when(lax.axis_index("s") == 0)
    def _():
        pltpu.sync_copy(tab_hbm, tab_v)

    plsc.subcore_barrier()
    idx_b = (idx0, idx1)
    rows_b = (rows0, rows1)
    s_idx = (s_idx0, s_idx1)
    s_out = (s_out0, s_out1)

    def idx_start(g, b):
        pltpu.async_copy(idx_hbm.at[pl.ds(w_base + g * SUP, SUP)],
                         idx_b[b], s_idx[b])

    def idx_wait(b):
        pltpu.make_async_copy(idx_hbm.at[pl.ds(0, SUP)],
                              idx_b[b], s_idx[b]).wait()

    def out_start(g, b):
        pltpu.async_copy(rows_b[b],
                         out_hbm.at[pl.ds(w_base + g * SUP, SUP), pl.ds(0, D)],
                         s_out[b])

    def out_wait(b):
        pltpu.make_async_copy(rows_b[b],
                              out_hbm.at[pl.ds(0, SUP), pl.ds(0, D)],
                              s_out[b]).wait()

    idx_start(0, 0)
    idx_start(1, 1)

    def pair(p, carry):
        for b in range(2):
            g = 2 * p + b

            @pl.when(p > 0)
            def _():
                out_wait(b)

            idx_wait(b)
            for j in range(GPC):
                pltpu.async_copy(
                    tab_v.at[idx_b[b].at[pl.ds(j * CHUNK, CHUNK)]],
                    rows_b[b].at[pl.ds(j * CHUNK, CHUNK)], s_gat)
            for j in range(GPC):
                pltpu.make_async_copy(
                    tab_v.at[idx_b[b].at[pl.ds(0, CHUNK)]],
                    rows_b[b].at[pl.ds(0, CHUNK)], s_gat).wait()

            @pl.when(g + 2 < NSUP)
            def _():
                idx_start(g + 2, b)

            out_start(g, b)
        return carry

    lax.fori_loop(0, NSUP // 2, pair, 0)
    out_wait(0)
    out_wait(1)


def kernel(onehot, emb_weight):
    flat = onehot.reshape(N)
    out = _emb_lookup(flat, emb_weight)
    return out[:, :D].reshape(R, C, D)


# speedup vs baseline: 2.1748x; 1.0020x over previous
import functools

import jax
import jax.numpy as jnp
from jax import lax
from jax.experimental import pallas as pl
from jax.experimental.pallas import tpu as pltpu
from jax.experimental.pallas import tpu_sc as plsc

R, C = 16384, 200
D = 64
N = R * C

NC, NS = 2, 16
NW = NC * NS
PER_W = N // NW
CHUNK = 128
SUP = 640
GPC = SUP // CHUNK
NSUP = PER_W // SUP

_mesh = plsc.VectorSubcoreMesh(core_axis_name="c", subcore_axis_name="s")


@functools.partial(
    pl.kernel,
    mesh=_mesh,
    out_type=jax.ShapeDtypeStruct((N, 128), jnp.float32),
    scratch_types=[
        pltpu.VMEM((SUP,), jnp.int32),
        pltpu.VMEM((SUP,), jnp.int32),
        pltpu.VMEM((SUP, D), jnp.float32),
        pltpu.VMEM((SUP, D), jnp.float32),
        pltpu.VMEM_SHARED((8, D), jnp.float32),
        pltpu.SemaphoreType.DMA,
        pltpu.SemaphoreType.DMA,
        pltpu.SemaphoreType.DMA,
        pltpu.SemaphoreType.DMA,
        pltpu.SemaphoreType.DMA,
    ],
    compiler_params=pltpu.CompilerParams(use_tc_tiling_on_sc=False),
)
def _emb_lookup(idx_hbm, tab_hbm, out_hbm, idx0, idx1, rows0, rows1, tab_v,
                s_idx0, s_idx1, s_gat, s_out0, s_out1):
    wid = lax.axis_index("s") * NC + lax.axis_index("c")
    w_base = wid * PER_W
    @pl.when(lax.axis_index("s") == 0)
    def _():
        pltpu.sync_copy(tab_hbm, tab_v)

    plsc.subcore_barrier()
    idx_b = (idx0, idx1)
    rows_b = (rows0, rows1)
    s_idx = (s_idx0, s_idx1)
    s_out = (s_out0, s_out1)

    def idx_start(g, b):
        pltpu.async_copy(idx_hbm.at[pl.ds(w_base + g * SUP, SUP)],
                         idx_b[b], s_idx[b])

    def idx_wait(b):
        pltpu.make_async_copy(idx_hbm.at[pl.ds(0, SUP)],
                              idx_b[b], s_idx[b]).wait()

    def out_start(g, b):
        pltpu.async_copy(rows_b[b],
                         out_hbm.at[pl.ds(w_base + g * SUP, SUP), pl.ds(0, D)],
                         s_out[b])

    def out_wait(b):
        pltpu.make_async_copy(rows_b[b],
                              out_hbm.at[pl.ds(0, SUP), pl.ds(0, D)],
                              s_out[b]).wait()

    idx_start(0, 0)
    idx_start(1, 1)

    def pair(p, carry):
        for b in range(2):
            g = 2 * p + b

            @pl.when(p > 0)
            def _():
                out_wait(b)

            idx_wait(b)
            for j in range(GPC):
                pltpu.async_copy(
                    tab_v.at[idx_b[b].at[pl.ds(j * CHUNK, CHUNK)]],
                    rows_b[b].at[pl.ds(j * CHUNK, CHUNK)], s_gat)
            for j in range(GPC):
                pltpu.make_async_copy(
                    tab_v.at[idx_b[b].at[pl.ds(0, CHUNK)]],
                    rows_b[b].at[pl.ds(0, CHUNK)], s_gat).wait()

            @pl.when(g + 2 < NSUP)
            def _():
                idx_start(g + 2, b)

            out_start(g, b)
        return carry

    lax.fori_loop(0, NSUP // 2, pair, 0)
    out_wait(0)
    out_wait(1)


def kernel(onehot, emb_weight):
    flat = onehot.reshape(N)
    out = _emb_lookup(flat, emb_weight)
    return out[:, :D].reshape(R, C, D)
